# baseline (speedup 1.0000x reference)
import jax
import jax.numpy as jnp
from jax import lax
from jax.experimental import pallas as pl
from jax.experimental.pallas import tpu as pltpu
from jax.experimental.pallas import tpu_sc as plsc

NC = 2
NS = 16
L = 16
NW = NC * NS
B = 16384
D = 64
V = 1000000
BLK = 128
BPW = B // NW
NBUF = 8
NB_I = 4
RS = 48


def _body(user_hbm, item_hbm, ut_hbm, it_hbm, w_hbm, out_hbm,
          uidx_v, iidx_v, w_v, out_v, ubuf, ibuf16, slv, sbuf, *sems):
    wid = lax.axis_index("s") * NC + lax.axis_index("c")
    sid = lax.axis_index("s")
    base = wid * BPW

    pltpu.sync_copy(user_hbm.at[pl.ds(base, BPW)],
                    uidx_v.at[pl.ds(0, BPW)])
    pltpu.sync_copy(item_hbm.at[pl.ds(base, BPW)],
                    iidx_v.at[pl.ds(0, BPW)])
    pltpu.sync_copy(w_hbm, w_v)

    sems_u = sems[:NBUF]
    sems_t = sems[NBUF:2 * NBUF]
    sems_i = sems[2 * NBUF:]
    lane = lax.iota(jnp.int32, L)
    wvecs = [w_v[pl.ds(c * L, L)] for c in range(D // L)]

    def get_scalar(ref, j):
        return ref[pl.ds(j, L)][0]

    def start_u(j, slot):
        u = get_scalar(uidx_v, j)
        ub = (u // BLK) * BLK
        pltpu.async_copy(ut_hbm.at[pl.ds(0, D), pl.ds(ub, BLK)],
                         ubuf.at[slot], sems_u[slot])

    def start_i16(j, slot):
        i = get_scalar(iidx_v, j)
        ib = (i // BLK) * BLK
        pltpu.async_copy(it_hbm.at[pl.ds(0, D - RS), pl.ds(ib, BLK)],
                         ibuf16.at[slot], sems_t[slot])

    def start_i(j, slot):
        i = get_scalar(iidx_v, j)
        ib = (i // BLK) * BLK
        pltpu.async_copy(it_hbm.at[pl.ds(D - RS, RS), pl.ds(ib, BLK)],
                         sbuf.at[sid, slot], sems_i[slot])

    def wait_u(slot):
        pltpu.make_async_copy(ut_hbm.at[pl.ds(0, D), pl.ds(0, BLK)],
                              ubuf.at[slot], sems_u[slot]).wait()

    def wait_i16(slot):
        pltpu.make_async_copy(it_hbm.at[pl.ds(0, D - RS), pl.ds(0, BLK)],
                              ibuf16.at[slot], sems_t[slot]).wait()

    def wait_i(slot):
        pltpu.make_async_copy(it_hbm.at[pl.ds(D - RS, RS), pl.ds(0, BLK)],
                              sbuf.at[sid, slot], sems_i[slot]).wait()

    def compute(j, slot, islot, vec):
        u = get_scalar(uidx_v, j)
        i = get_scalar(iidx_v, j)
        c16 = (i % BLK) // L * L
        pltpu.sync_copy(sbuf.at[sid, islot, pl.ds(0, RS), pl.ds(c16, L)],
                        slv.at[pl.ds(0, RS), pl.ds(slot * L, L)])
        ucol = jnp.zeros((L,), jnp.int32) + (u % BLK)
        icol = jnp.zeros((L,), jnp.int32) + (slot * L + (i % L))
        sl = jnp.full((L,), slot, jnp.int32)
        acc = jnp.zeros((L,), jnp.float32)
        icolf = jnp.zeros((L,), jnp.int32) + (i % BLK)
        for c in range(D // L):
            rows = c * L + lane
            uv = plsc.load_gather(ubuf, [sl, rows, ucol])
            if c == 0:
                iv = plsc.load_gather(ibuf16, [sl, lane, icolf])
            else:
                iv = plsc.load_gather(slv, [(c - 1) * L + lane, icol])
            acc = acc + uv * iv * wvecs[c]
        s_val = jnp.sum(acc)
        return jnp.where(lane == (j % L), s_val, vec)

    for j in range(NBUF - 1):
        start_u(j, j)
        start_i16(j, j)
    for j in range(NB_I - 1):
        start_i(j, j)

    def outer(k, vec):
        for s in range(NBUF):
            j = k * NBUF + s
            jju = j + NBUF - 1
            jji = j + NB_I - 1

            @pl.when(jju < BPW)
            def _():
                start_u(jju, (s + NBUF - 1) % NBUF)
                start_i16(jju, (s + NBUF - 1) % NBUF)

            @pl.when(jji < BPW)
            def _():
                start_i(jji, (s + NB_I - 1) % NB_I)

            wait_u(s)
            wait_i16(s)
            wait_i(s % NB_I)
            vec = compute(j, s, s % NB_I, vec)

            @pl.when(j % L == L - 1)
            def _():
                out_v[pl.ds((j // L) * L, L)] = vec
        return vec

    lax.fori_loop(0, BPW // NBUF, outer, jnp.zeros((L,), jnp.float32))
    pltpu.sync_copy(out_v, out_hbm.at[pl.ds(base, BPW)])


@jax.jit
def _run(user, item, ut, it, w):
    mesh = plsc.VectorSubcoreMesh(core_axis_name="c", subcore_axis_name="s",
                                  num_cores=NC, num_subcores=NS)
    f = pl.kernel(
        _body,
        out_type=jax.ShapeDtypeStruct((B,), jnp.float32),
        mesh=mesh,
        scratch_types=[
            pltpu.VMEM((BPW + L,), jnp.int32),
            pltpu.VMEM((BPW + L,), jnp.int32),
            pltpu.VMEM((D,), jnp.float32),
            pltpu.VMEM((BPW,), jnp.float32),
            pltpu.VMEM((NBUF, D, BLK), jnp.float32),
            pltpu.VMEM((NBUF, D - RS, BLK), jnp.float32),
            pltpu.VMEM((RS, NBUF * L), jnp.float32),
            pltpu.VMEM_SHARED((NS, NB_I, RS, BLK), jnp.float32),
        ] + [pltpu.SemaphoreType.DMA] * (2 * NBUF + NB_I),
        compiler_params=pltpu.CompilerParams(needs_layout_passes=False,
                                             use_tc_tiling_on_sc=True),
    )
    return f(user, item, ut, it, w)


def kernel(user, item, embed_user_weight, embed_item_weight, predict_layer):
    user = user.astype(jnp.int32)
    item = item.astype(jnp.int32)
    ut = embed_user_weight.T
    it = embed_item_weight.T
    w = predict_layer.reshape(-1).astype(jnp.float32)
    return _run(user, item, ut, it, w)


# speedup vs baseline: 1.0054x; 1.0054x over previous
import jax
import jax.numpy as jnp
from jax import lax
from jax.experimental import pallas as pl
from jax.experimental.pallas import tpu as pltpu
from jax.experimental.pallas import tpu_sc as plsc

NC = 2
NS = 16
L = 16
NW = NC * NS
B = 16384
D = 64
V = 1000000
BLK = 128
BPW = B // NW
NBUF = 8
NB_I = 4
RS = 48


def _body(user_hbm, item_hbm, ut_hbm, it_hbm, w_hbm, out_hbm,
          uidx_v, iidx_v, w_v, out_v, ubuf, ibuf16, slv, sbuf, *sems):
    wid = lax.axis_index("s") * NC + lax.axis_index("c")
    sid = lax.axis_index("s")
    base = wid * BPW

    pltpu.sync_copy(user_hbm.at[pl.ds(base, BPW)],
                    uidx_v.at[pl.ds(0, BPW)])
    pltpu.sync_copy(item_hbm.at[pl.ds(base, BPW)],
                    iidx_v.at[pl.ds(0, BPW)])
    pltpu.sync_copy(w_hbm, w_v)

    sems_u = sems[:NBUF]
    sems_t = sems[NBUF:2 * NBUF]
    sems_i = sems[2 * NBUF:2 * NBUF + NB_I]
    sems_s = sems[2 * NBUF + NB_I:]
    lane = lax.iota(jnp.int32, L)
    wvecs = [w_v[pl.ds(c * L, L)] for c in range(D // L)]

    def get_scalar(ref, j):
        return ref[pl.ds(j, L)][0]

    def start_u(j, slot):
        u = get_scalar(uidx_v, j)
        ub = (u // BLK) * BLK
        pltpu.async_copy(ut_hbm.at[pl.ds(0, D), pl.ds(ub, BLK)],
                         ubuf.at[slot], sems_u[slot])

    def start_i16(j, slot):
        i = get_scalar(iidx_v, j)
        ib = (i // BLK) * BLK
        pltpu.async_copy(it_hbm.at[pl.ds(0, D - RS), pl.ds(ib, BLK)],
                         ibuf16.at[slot], sems_t[slot])

    def start_i(j, slot):
        i = get_scalar(iidx_v, j)
        ib = (i // BLK) * BLK
        pltpu.async_copy(it_hbm.at[pl.ds(D - RS, RS), pl.ds(ib, BLK)],
                         sbuf.at[sid, slot], sems_i[slot])

    def wait_u(slot):
        pltpu.make_async_copy(ut_hbm.at[pl.ds(0, D), pl.ds(0, BLK)],
                              ubuf.at[slot], sems_u[slot]).wait()

    def wait_i16(slot):
        pltpu.make_async_copy(it_hbm.at[pl.ds(0, D - RS), pl.ds(0, BLK)],
                              ibuf16.at[slot], sems_t[slot]).wait()

    def wait_i(slot):
        pltpu.make_async_copy(it_hbm.at[pl.ds(D - RS, RS), pl.ds(0, BLK)],
                              sbuf.at[sid, slot], sems_i[slot]).wait()

    def start_sliver(j, slot, islot):
        i = get_scalar(iidx_v, j)
        c16 = (i % BLK) // L * L
        pltpu.async_copy(sbuf.at[sid, islot, pl.ds(0, RS), pl.ds(c16, L)],
                         slv.at[pl.ds(0, RS), pl.ds(slot * L, L)],
                         sems_s[slot])

    def wait_sliver(slot):
        pltpu.make_async_copy(sbuf.at[sid, 0, pl.ds(0, RS), pl.ds(0, L)],
                              slv.at[pl.ds(0, RS), pl.ds(slot * L, L)],
                              sems_s[slot]).wait()

    def compute(j, slot, vec):
        u = get_scalar(uidx_v, j)
        i = get_scalar(iidx_v, j)
        ucol = jnp.zeros((L,), jnp.int32) + (u % BLK)
        icol = jnp.zeros((L,), jnp.int32) + (slot * L + (i % L))
        sl = jnp.full((L,), slot, jnp.int32)
        acc = jnp.zeros((L,), jnp.float32)
        icolf = jnp.zeros((L,), jnp.int32) + (i % BLK)
        for c in range(D // L):
            rows = c * L + lane
            uv = plsc.load_gather(ubuf, [sl, rows, ucol])
            if c == 0:
                iv = plsc.load_gather(ibuf16, [sl, lane, icolf])
            else:
                iv = plsc.load_gather(slv, [(c - 1) * L + lane, icol])
            acc = acc + uv * iv * wvecs[c]
        s_val = jnp.sum(acc)
        return jnp.where(lane == (j % L), s_val, vec)

    for j in range(NBUF - 1):
        start_u(j, j)
        start_i16(j, j)
    for j in range(NB_I - 1):
        start_i(j, j)
    wait_i(0)
    start_sliver(0, 0, 0)

    def outer(k, vec):
        for s in range(NBUF):
            j = k * NBUF + s
            jju = j + NBUF - 1
            jji = j + NB_I - 1

            @pl.when(jju < BPW)
            def _():
                start_u(jju, (s + NBUF - 1) % NBUF)
                start_i16(jju, (s + NBUF - 1) % NBUF)

            @pl.when(jji < BPW)
            def _():
                start_i(jji, (s + NB_I - 1) % NB_I)

            @pl.when(j + 1 < BPW)
            def _():
                wait_i((s + 1) % NB_I)
                start_sliver(j + 1, (s + 1) % NBUF, (s + 1) % NB_I)

            wait_u(s)
            wait_i16(s)
            wait_sliver(s)
            vec = compute(j, s, vec)

            @pl.when(j % L == L - 1)
            def _():
                out_v[pl.ds((j // L) * L, L)] = vec
        return vec

    lax.fori_loop(0, BPW // NBUF, outer, jnp.zeros((L,), jnp.float32))
    pltpu.sync_copy(out_v, out_hbm.at[pl.ds(base, BPW)])


@jax.jit
def _run(user, item, ut, it, w):
    mesh = plsc.VectorSubcoreMesh(core_axis_name="c", subcore_axis_name="s",
                                  num_cores=NC, num_subcores=NS)
    f = pl.kernel(
        _body,
        out_type=jax.ShapeDtypeStruct((B,), jnp.float32),
        mesh=mesh,
        scratch_types=[
            pltpu.VMEM((BPW + L,), jnp.int32),
            pltpu.VMEM((BPW + L,), jnp.int32),
            pltpu.VMEM((D,), jnp.float32),
            pltpu.VMEM((BPW,), jnp.float32),
            pltpu.VMEM((NBUF, D, BLK), jnp.float32),
            pltpu.VMEM((NBUF, D - RS, BLK), jnp.float32),
            pltpu.VMEM((RS, NBUF * L), jnp.float32),
            pltpu.VMEM_SHARED((NS, NB_I, RS, BLK), jnp.float32),
        ] + [pltpu.SemaphoreType.DMA] * (3 * NBUF + NB_I),
        compiler_params=pltpu.CompilerParams(needs_layout_passes=False,
                                             use_tc_tiling_on_sc=True),
    )
    return f(user, item, ut, it, w)


def kernel(user, item, embed_user_weight, embed_item_weight, predict_layer):
    user = user.astype(jnp.int32)
    item = item.astype(jnp.int32)
    ut = embed_user_weight.T
    it = embed_item_weight.T
    w = predict_layer.reshape(-1).astype(jnp.float32)
    return _run(user, item, ut, it, w)


# speedup vs baseline: 1.0305x; 1.0250x over previous
import jax
import jax.numpy as jnp
from jax import lax
from jax.experimental import pallas as pl
from jax.experimental.pallas import tpu as pltpu
from jax.experimental.pallas import tpu_sc as plsc

NC = 2
NS = 16
L = 16
NW = NC * NS
B = 16384
D = 64
V = 1000000
BLK = 128
BPW = B // NW
NBUF = 8
NB_I = 4


def _body(user_hbm, item_hbm, ut_hbm, it_hbm, w_hbm, out_hbm,
          uidx_v, iidx_v, w_v, out_v, ubuf, slv, sbuf, *sems):
    wid = lax.axis_index("s") * NC + lax.axis_index("c")
    sid = lax.axis_index("s")
    base = wid * BPW

    pltpu.sync_copy(user_hbm.at[pl.ds(base, BPW)],
                    uidx_v.at[pl.ds(0, BPW)])
    pltpu.sync_copy(item_hbm.at[pl.ds(base, BPW)],
                    iidx_v.at[pl.ds(0, BPW)])
    pltpu.sync_copy(w_hbm, w_v)

    sems_u = sems[:NBUF]
    sems_i = sems[NBUF:NBUF + NB_I]
    sems_s = sems[NBUF + NB_I:]
    lane = lax.iota(jnp.int32, L)
    wvecs = [w_v[pl.ds(c * L, L)] for c in range(D // L)]

    def get_scalar(ref, j):
        return ref[pl.ds(j, L)][0]

    def start_u(j, slot):
        u = get_scalar(uidx_v, j)
        ub = (u // BLK) * BLK
        pltpu.async_copy(ut_hbm.at[pl.ds(0, D), pl.ds(ub, BLK)],
                         ubuf.at[slot], sems_u[slot])

    def start_i(j, slot):
        i = get_scalar(iidx_v, j)
        ib = (i // BLK) * BLK
        pltpu.async_copy(it_hbm.at[pl.ds(0, D), pl.ds(ib, BLK)],
                         sbuf.at[sid, slot], sems_i[slot])

    def wait_u(slot):
        pltpu.make_async_copy(ut_hbm.at[pl.ds(0, D), pl.ds(0, BLK)],
                              ubuf.at[slot], sems_u[slot]).wait()

    def wait_i(slot):
        pltpu.make_async_copy(it_hbm.at[pl.ds(0, D), pl.ds(0, BLK)],
                              sbuf.at[sid, slot], sems_i[slot]).wait()

    def start_sliver(j, slot, islot):
        i = get_scalar(iidx_v, j)
        c16 = (i % BLK) // L * L
        pltpu.async_copy(sbuf.at[sid, islot, pl.ds(0, D), pl.ds(c16, L)],
                         slv.at[pl.ds(0, D), pl.ds(slot * L, L)],
                         sems_s[slot])

    def wait_sliver(slot):
        pltpu.make_async_copy(sbuf.at[sid, 0, pl.ds(0, D), pl.ds(0, L)],
                              slv.at[pl.ds(0, D), pl.ds(slot * L, L)],
                              sems_s[slot]).wait()

    def compute(j, slot, vec):
        u = get_scalar(uidx_v, j)
        i = get_scalar(iidx_v, j)
        ucol = jnp.zeros((L,), jnp.int32) + (u % BLK)
        icol = jnp.zeros((L,), jnp.int32) + (slot * L + (i % L))
        sl = jnp.full((L,), slot, jnp.int32)
        acc = jnp.zeros((L,), jnp.float32)
        for c in range(D // L):
            rows = c * L + lane
            uv = plsc.load_gather(ubuf, [sl, rows, ucol])
            iv = plsc.load_gather(slv, [rows, icol])
            acc = acc + uv * iv * wvecs[c]
        s_val = jnp.sum(acc)
        return jnp.where(lane == (j % L), s_val, vec)

    for j in range(NBUF - 1):
        start_u(j, j)
    for j in range(NB_I - 1):
        start_i(j, j)
    wait_i(0)
    start_sliver(0, 0, 0)

    def outer(k, vec):
        for s in range(NBUF):
            j = k * NBUF + s
            jju = j + NBUF - 1
            jji = j + NB_I - 1

            @pl.when(jju < BPW)
            def _():
                start_u(jju, (s + NBUF - 1) % NBUF)

            @pl.when(jji < BPW)
            def _():
                start_i(jji, (s + NB_I - 1) % NB_I)

            @pl.when(j + 1 < BPW)
            def _():
                wait_i((s + 1) % NB_I)
                start_sliver(j + 1, (s + 1) % NBUF, (s + 1) % NB_I)

            wait_u(s)
            wait_sliver(s)
            vec = compute(j, s, vec)

            @pl.when(j % L == L - 1)
            def _():
                out_v[pl.ds((j // L) * L, L)] = vec
        return vec

    lax.fori_loop(0, BPW // NBUF, outer, jnp.zeros((L,), jnp.float32))
    pltpu.sync_copy(out_v, out_hbm.at[pl.ds(base, BPW)])


@jax.jit
def _run(user, item, ut, it, w):
    mesh = plsc.VectorSubcoreMesh(core_axis_name="c", subcore_axis_name="s",
                                  num_cores=NC, num_subcores=NS)
    f = pl.kernel(
        _body,
        out_type=jax.ShapeDtypeStruct((B,), jnp.float32),
        mesh=mesh,
        scratch_types=[
            pltpu.VMEM((BPW + L,), jnp.int32),
            pltpu.VMEM((BPW + L,), jnp.int32),
            pltpu.VMEM((D,), jnp.float32),
            pltpu.VMEM((BPW,), jnp.float32),
            pltpu.VMEM((NBUF, D, BLK), jnp.float32),
            pltpu.VMEM((D, NBUF * L), jnp.float32),
            pltpu.VMEM_SHARED((NS, NB_I, D, BLK), jnp.float32),
        ] + [pltpu.SemaphoreType.DMA] * (2 * NBUF + NB_I),
        compiler_params=pltpu.CompilerParams(needs_layout_passes=False,
                                             use_tc_tiling_on_sc=True),
    )
    return f(user, item, ut, it, w)


def kernel(user, item, embed_user_weight, embed_item_weight, predict_layer):
    user = user.astype(jnp.int32)
    item = item.astype(jnp.int32)
    ut = embed_user_weight.T
    it = embed_item_weight.T
    w = predict_layer.reshape(-1).astype(jnp.float32)
    return _run(user, item, ut, it, w)


# speedup vs baseline: 1.0430x; 1.0121x over previous
import jax
import jax.numpy as jnp
from jax import lax
from jax.experimental import pallas as pl
from jax.experimental.pallas import tpu as pltpu
from jax.experimental.pallas import tpu_sc as plsc

NC = 2
NS = 16
L = 16
NW = NC * NS
B = 16384
D = 64
V = 1000000
BLK = 128
BPW = B // NW
NBUF = 8
NB_I = 4


def _body(user_hbm, item_hbm, ut_hbm, it_hbm, w_hbm, out_hbm,
          uidx_v, iidx_v, w_v, out_v, ubuf, slv, sbuf, *sems):
    wid = lax.axis_index("s") * NC + lax.axis_index("c")
    sid = lax.axis_index("s")
    base = wid * BPW

    pltpu.sync_copy(user_hbm.at[pl.ds(base, BPW)],
                    uidx_v.at[pl.ds(0, BPW)])
    pltpu.sync_copy(item_hbm.at[pl.ds(base, BPW)],
                    iidx_v.at[pl.ds(0, BPW)])
    pltpu.sync_copy(w_hbm, w_v)

    sems_u = sems[:NBUF]
    sems_i = sems[NBUF:]
    lane = lax.iota(jnp.int32, L)
    wvecs = [w_v[pl.ds(c * L, L)] for c in range(D // L)]

    def get_scalar(ref, j):
        return ref[pl.ds(j, L)][0]

    def start_u(j, slot):
        u = get_scalar(uidx_v, j)
        ub = (u // BLK) * BLK
        pltpu.async_copy(ut_hbm.at[pl.ds(0, D), pl.ds(ub, BLK)],
                         ubuf.at[slot], sems_u[slot])

    def start_i(j, slot):
        i = get_scalar(iidx_v, j)
        ib = (i // BLK) * BLK
        pltpu.async_copy(it_hbm.at[pl.ds(0, D), pl.ds(ib, BLK)],
                         sbuf.at[sid, slot], sems_i[slot])

    def wait_u(slot):
        pltpu.make_async_copy(ut_hbm.at[pl.ds(0, D), pl.ds(0, BLK)],
                              ubuf.at[slot], sems_u[slot]).wait()

    def wait_i(slot):
        pltpu.make_async_copy(it_hbm.at[pl.ds(0, D), pl.ds(0, BLK)],
                              sbuf.at[sid, slot], sems_i[slot]).wait()

    def compute(j, slot, islot, vec):
        u = get_scalar(uidx_v, j)
        i = get_scalar(iidx_v, j)
        c16 = (i % BLK) // L * L
        pltpu.sync_copy(sbuf.at[sid, islot, pl.ds(0, D), pl.ds(c16, L)],
                        slv.at[pl.ds(0, D), pl.ds(slot * L, L)])
        ucol = jnp.zeros((L,), jnp.int32) + (u % BLK)
        icol = jnp.zeros((L,), jnp.int32) + (slot * L + (i % L))
        sl = jnp.full((L,), slot, jnp.int32)
        acc = jnp.zeros((L,), jnp.float32)
        for c in range(D // L):
            rows = c * L + lane
            uv = plsc.load_gather(ubuf, [sl, rows, ucol])
            iv = plsc.load_gather(slv, [rows, icol])
            acc = acc + uv * iv * wvecs[c]
        s_val = jnp.sum(acc)
        return jnp.where(lane == (j % L), s_val, vec)

    for j in range(NBUF - 1):
        start_u(j, j)
    for j in range(NB_I - 1):
        start_i(j, j)

    def outer(k, vec):
        for s in range(NBUF):
            j = k * NBUF + s
            jju = j + NBUF - 1
            jji = j + NB_I - 1

            @pl.when(jju < BPW)
            def _():
                start_u(jju, (s + NBUF - 1) % NBUF)

            @pl.when(jji < BPW)
            def _():
                start_i(jji, (s + NB_I - 1) % NB_I)

            wait_u(s)
            wait_i(s % NB_I)
            vec = compute(j, s, s % NB_I, vec)

            @pl.when(j % L == L - 1)
            def _():
                out_v[pl.ds((j // L) * L, L)] = vec
        return vec

    lax.fori_loop(0, BPW // NBUF, outer, jnp.zeros((L,), jnp.float32))
    pltpu.sync_copy(out_v, out_hbm.at[pl.ds(base, BPW)])


@jax.jit
def _run(user, item, ut, it, w):
    mesh = plsc.VectorSubcoreMesh(core_axis_name="c", subcore_axis_name="s",
                                  num_cores=NC, num_subcores=NS)
    f = pl.kernel(
        _body,
        out_type=jax.ShapeDtypeStruct((B,), jnp.float32),
        mesh=mesh,
        scratch_types=[
            pltpu.VMEM((BPW + L,), jnp.int32),
            pltpu.VMEM((BPW + L,), jnp.int32),
            pltpu.VMEM((D,), jnp.float32),
            pltpu.VMEM((BPW,), jnp.float32),
            pltpu.VMEM((NBUF, D, BLK), jnp.float32),
            pltpu.VMEM((D, NBUF * L), jnp.float32),
            pltpu.VMEM_SHARED((NS, NB_I, D, BLK), jnp.float32),
        ] + [pltpu.SemaphoreType.DMA] * (NBUF + NB_I),
        compiler_params=pltpu.CompilerParams(needs_layout_passes=False,
                                             use_tc_tiling_on_sc=True),
    )
    return f(user, item, ut, it, w)


def kernel(user, item, embed_user_weight, embed_item_weight, predict_layer):
    user = user.astype(jnp.int32)
    item = item.astype(jnp.int32)
    ut = embed_user_weight.T
    it = embed_item_weight.T
    w = predict_layer.reshape(-1).astype(jnp.float32)
    return _run(user, item, ut, it, w)
